# baseline (speedup 1.0000x reference)
import jax
import jax.numpy as jnp
from jax.experimental import pallas as pl
from jax.experimental.pallas import tpu as pltpu


def _body(order, rank, x_ref, f_ref, o_ref):
    y = jnp.dot(x_ref[...], f_ref[...], preferred_element_type=jnp.float32)
    acc = y[:, 0:rank]
    for a in range(1, order):
        acc = acc * y[:, a * rank:(a + 1) * rank]
    o_ref[:, :] = jnp.sum(acc, axis=1, keepdims=True)


def kernel(indices_tensor, factors):
    B, order, M = indices_tensor.shape
    R = factors.shape[-1]
    BS = 512

    x = indices_tensor.reshape(B, order * M)
    f_bd = jax.scipy.linalg.block_diag(*factors)

    import functools
    body = functools.partial(_body, order, R)

    out = pl.pallas_call(
        body,
        grid=(B // BS,),
        in_specs=[
            pl.BlockSpec((BS, order * M), lambda i: (i, 0)),
            pl.BlockSpec((order * M, order * R), lambda i: (0, 0)),
        ],
        out_specs=pl.BlockSpec((BS, 1), lambda i: (i, 0)),
        out_shape=jax.ShapeDtypeStruct((B, 1), jnp.float32),
        compiler_params=pltpu.CompilerParams(
            dimension_semantics=("arbitrary",),
        ),
    )(x, f_bd)
    return out[:, 0]
